# Initial kernel scaffold; baseline (speedup 1.0000x reference)
import functools

import jax
import jax.numpy as jnp
from jax import lax
from jax.experimental import pallas as pl
from jax.experimental.pallas import tpu as pltpu
from jax.experimental.pallas import tpu_sc as plsc

_NSPEC = 119
_TBL = 128
_NC = 2
_NS = 16
_NW = _NC * _NS
_L = 16
_CHUNK = 8192


def _body(n, chunk, x_hbm, z_hbm, scale_hbm, shift_hbm, out_hbm,
          scale_v, shift_v, xb, zb, ob):
    c = lax.axis_index("c")
    s = lax.axis_index("s")
    wid = s * _NC + c
    per_w = n // _NW
    base = wid * per_w
    pltpu.sync_copy(scale_hbm, scale_v)
    pltpu.sync_copy(shift_hbm, shift_v)

    def chunk_body(ci, carry):
        off = base + ci * chunk
        pltpu.sync_copy(x_hbm.at[pl.ds(off, chunk)], xb)
        pltpu.sync_copy(z_hbm.at[pl.ds(off, chunk)], zb)

        def vec_body(i, carry2):
            sl = pl.ds(i * _L, _L)
            zv = zb[sl]
            xv = xb[sl]
            sv = plsc.load_gather(scale_v, [zv])
            tv = plsc.load_gather(shift_v, [zv])
            ob[sl] = sv * xv + tv
            return carry2

        lax.fori_loop(0, chunk // _L, vec_body, 0, unroll=4)
        pltpu.sync_copy(ob, out_hbm.at[pl.ds(off, chunk)])
        return carry

    lax.fori_loop(0, per_w // chunk, chunk_body, 0)


@functools.partial(jax.jit, static_argnames=("interpret",))
def kernel(x, Z, scale, shift, interpret=False):
    x = x.astype(jnp.float32)
    n = x.shape[0]
    per_w = n // _NW
    chunk = min(_CHUNK, per_w)
    scale_p = jnp.pad(scale.astype(jnp.float32), (0, _TBL - _NSPEC))
    shift_p = jnp.pad(shift.astype(jnp.float32), (0, _TBL - _NSPEC))
    mesh = plsc.VectorSubcoreMesh(core_axis_name="c", subcore_axis_name="s",
                                  num_cores=_NC, num_subcores=_NS)
    f = pl.kernel(
        functools.partial(_body, n, chunk),
        out_type=jax.ShapeDtypeStruct((n,), jnp.float32),
        mesh=mesh,
        interpret=interpret,
        scratch_types=[
            pltpu.VMEM((_TBL,), jnp.float32),
            pltpu.VMEM((_TBL,), jnp.float32),
            pltpu.VMEM((chunk,), jnp.float32),
            pltpu.VMEM((chunk,), jnp.int32),
            pltpu.VMEM((chunk,), jnp.float32),
        ],
    )
    return f(x, Z.astype(jnp.int32), scale_p, shift_p)


# speedup vs baseline: 523.5726x; 523.5726x over previous
import functools

import jax
import jax.numpy as jnp
from jax import lax
from jax.experimental import pallas as pl
from jax.experimental.pallas import tpu as pltpu
from jax.experimental.pallas import tpu_sc as plsc

_NSPEC = 119
_TBL = 128
_NC = 2
_NS = 16
_NW = _NC * _NS
_L = 16
_CHUNK = 8192


def _body(n, chunk, x_hbm, z_hbm, scale_hbm, shift_hbm, out_hbm,
          scale_v, shift_v, xb, zb, ob):
    c = lax.axis_index("c")
    s = lax.axis_index("s")
    wid = s * _NC + c
    per_w = n // _NW
    base = wid * per_w
    pltpu.sync_copy(scale_hbm, scale_v)
    pltpu.sync_copy(shift_hbm, shift_v)

    def chunk_body(ci, carry):
        off = base + ci * chunk
        pltpu.sync_copy(x_hbm.at[pl.ds(off, chunk)], xb)
        pltpu.sync_copy(z_hbm.at[pl.ds(off, chunk)], zb)

        def vec_body(i, carry2):
            sl = pl.ds(i * _L, _L)
            zv = zb[sl]
            xv = xb[sl]
            sv = plsc.load_gather(scale_v, [zv])
            tv = plsc.load_gather(shift_v, [zv])
            ob[sl] = sv * xv + tv
            return carry2

        lax.fori_loop(0, chunk // _L, vec_body, 0, unroll=4)
        pltpu.sync_copy(ob, out_hbm.at[pl.ds(off, chunk)])
        return carry

    lax.fori_loop(0, per_w // chunk, chunk_body, 0)


@functools.partial(jax.jit, static_argnames=("interpret",))
def kernel(x, Z, scale, shift, interpret=False):
    x = x.astype(jnp.float32)
    n = x.shape[0]
    per_w = n // _NW
    chunk = min(_CHUNK, per_w)
    scale_p = jnp.pad(scale.astype(jnp.float32), (0, _TBL - _NSPEC))
    shift_p = jnp.pad(shift.astype(jnp.float32), (0, _TBL - _NSPEC))
    mesh = plsc.VectorSubcoreMesh(core_axis_name="c", subcore_axis_name="s",
                                  num_cores=_NC, num_subcores=_NS)
    f = pl.kernel(
        functools.partial(_body, n, chunk),
        out_type=jax.ShapeDtypeStruct((n,), jnp.float32),
        mesh=mesh,
        interpret=interpret,
        compiler_params=pltpu.CompilerParams(needs_layout_passes=False),
        scratch_types=[
            pltpu.VMEM((_TBL,), jnp.float32),
            pltpu.VMEM((_TBL,), jnp.float32),
            pltpu.VMEM((chunk,), jnp.float32),
            pltpu.VMEM((chunk,), jnp.int32),
            pltpu.VMEM((chunk,), jnp.float32),
        ],
    )
    return f(x, Z.astype(jnp.int32), scale_p, shift_p)


# speedup vs baseline: 662.8097x; 1.2659x over previous
import functools

import jax
import jax.numpy as jnp
from jax import lax
from jax.experimental import pallas as pl
from jax.experimental.pallas import tpu as pltpu
from jax.experimental.pallas import tpu_sc as plsc

_NSPEC = 119
_TBL = 128
_NC = 2
_NS = 16
_NW = _NC * _NS
_L = 16
_CHUNK = 8192


def _body(n, chunk, x_hbm, z_hbm, scale_hbm, shift_hbm, out_hbm,
          scale_v, shift_v, xb0, xb1, zb0, zb1, ob0, ob1,
          sx0, sx1, sz0, sz1, so0, so1):
    c = lax.axis_index("c")
    s = lax.axis_index("s")
    wid = s * _NC + c
    per_w = n // _NW
    base = wid * per_w
    nchunk = per_w // chunk
    pltpu.sync_copy(scale_hbm, scale_v)
    pltpu.sync_copy(shift_hbm, shift_v)

    xb = (xb0, xb1)
    zb = (zb0, zb1)
    ob = (ob0, ob1)
    sx = (sx0, sx1)
    sz = (sz0, sz1)
    so = (so0, so1)

    def start_in(ci):
        off = base + ci * chunk
        b = ci % 2
        return (pltpu.async_copy(x_hbm.at[pl.ds(off, chunk)], xb[b], sx[b]),
                pltpu.async_copy(z_hbm.at[pl.ds(off, chunk)], zb[b], sz[b]))

    in_copies = [None] * nchunk
    out_copies = [None] * nchunk
    in_copies[0] = start_in(0)
    for ci in range(nchunk):
        b = ci % 2
        if ci + 1 < nchunk:
            in_copies[ci + 1] = start_in(ci + 1)
        cx, cz = in_copies[ci]
        cx.wait()
        cz.wait()
        if ci >= 2:
            out_copies[ci - 2].wait()

        def vec_body(i, carry, b=b):
            sl = pl.ds(i * _L, _L)
            zv = zb[b][sl]
            xv = xb[b][sl]
            sv = plsc.load_gather(scale_v, [zv])
            tv = plsc.load_gather(shift_v, [zv])
            ob[b][sl] = sv * xv + tv
            return carry

        lax.fori_loop(0, chunk // _L, vec_body, 0, unroll=8)
        off = base + ci * chunk
        out_copies[ci] = pltpu.async_copy(
            ob[b], out_hbm.at[pl.ds(off, chunk)], so[b])
    if nchunk >= 2:
        out_copies[nchunk - 2].wait()
    out_copies[nchunk - 1].wait()


@functools.partial(jax.jit, static_argnames=("interpret",))
def kernel(x, Z, scale, shift, interpret=False):
    x = x.astype(jnp.float32)
    n = x.shape[0]
    per_w = n // _NW
    chunk = min(_CHUNK, per_w)
    scale_p = jnp.pad(scale.astype(jnp.float32), (0, _TBL - _NSPEC))
    shift_p = jnp.pad(shift.astype(jnp.float32), (0, _TBL - _NSPEC))
    mesh = plsc.VectorSubcoreMesh(core_axis_name="c", subcore_axis_name="s",
                                  num_cores=_NC, num_subcores=_NS)
    f = pl.kernel(
        functools.partial(_body, n, chunk),
        out_type=jax.ShapeDtypeStruct((n,), jnp.float32),
        mesh=mesh,
        interpret=interpret,
        compiler_params=pltpu.CompilerParams(needs_layout_passes=False),
        scratch_types=[
            pltpu.VMEM((_TBL,), jnp.float32),
            pltpu.VMEM((_TBL,), jnp.float32),
            pltpu.VMEM((chunk,), jnp.float32),
            pltpu.VMEM((chunk,), jnp.float32),
            pltpu.VMEM((chunk,), jnp.int32),
            pltpu.VMEM((chunk,), jnp.int32),
            pltpu.VMEM((chunk,), jnp.float32),
            pltpu.VMEM((chunk,), jnp.float32),
            pltpu.SemaphoreType.DMA,
            pltpu.SemaphoreType.DMA,
            pltpu.SemaphoreType.DMA,
            pltpu.SemaphoreType.DMA,
            pltpu.SemaphoreType.DMA,
            pltpu.SemaphoreType.DMA,
        ],
    )
    return f(x, Z.astype(jnp.int32), scale_p, shift_p)


# speedup vs baseline: 1585.9324x; 2.3927x over previous
import functools

import jax
import jax.numpy as jnp
from jax import lax
from jax.experimental import pallas as pl
from jax.experimental.pallas import tpu as pltpu
from jax.experimental.pallas import tpu_sc as plsc

_NSPEC = 119
_TBL = 128
_NC = 2
_NS = 16
_NW = _NC * _NS
_L = 16
_CHUNK = 8192


def _body(n, chunk, x_hbm, z_hbm, scale_hbm, shift_hbm, out_hbm,
          scale_v, shift_v, xb0, xb1, zb0, zb1, ob0, ob1,
          sx0, sx1, sz0, sz1, so0, so1):
    c = lax.axis_index("c")
    s = lax.axis_index("s")
    wid = s * _NC + c
    per_w = n // _NW
    base = wid * per_w
    nchunk = per_w // chunk
    pltpu.sync_copy(scale_hbm, scale_v)
    pltpu.sync_copy(shift_hbm, shift_v)

    xb = (xb0, xb1)
    zb = (zb0, zb1)
    ob = (ob0, ob1)
    sx = (sx0, sx1)
    sz = (sz0, sz1)
    so = (so0, so1)

    def start_in(ci):
        off = base + ci * chunk
        b = ci % 2
        return (pltpu.async_copy(x_hbm.at[pl.ds(off, chunk)], xb[b], sx[b]),
                pltpu.async_copy(z_hbm.at[pl.ds(off, chunk)], zb[b], sz[b]))

    in_copies = [None] * nchunk
    out_copies = [None] * nchunk
    in_copies[0] = start_in(0)
    for ci in range(nchunk):
        b = ci % 2
        if ci + 1 < nchunk:
            in_copies[ci + 1] = start_in(ci + 1)
        cx, cz = in_copies[ci]
        cx.wait()
        cz.wait()
        if ci >= 2:
            out_copies[ci - 2].wait()

        @plsc.parallel_loop(0, chunk, _L, unroll=8)
        def vec_body(i, b=b):
            i = pl.multiple_of(i, _L)
            sl = pl.ds(i, _L)
            zv = zb[b][sl]
            xv = xb[b][sl]
            sv = plsc.load_gather(scale_v, [zv])
            tv = plsc.load_gather(shift_v, [zv])
            ob[b][sl] = sv * xv + tv
        off = base + ci * chunk
        out_copies[ci] = pltpu.async_copy(
            ob[b], out_hbm.at[pl.ds(off, chunk)], so[b])
    if nchunk >= 2:
        out_copies[nchunk - 2].wait()
    out_copies[nchunk - 1].wait()


@functools.partial(jax.jit, static_argnames=("interpret",))
def kernel(x, Z, scale, shift, interpret=False):
    x = x.astype(jnp.float32)
    n = x.shape[0]
    per_w = n // _NW
    chunk = min(_CHUNK, per_w)
    scale_p = jnp.pad(scale.astype(jnp.float32), (0, _TBL - _NSPEC))
    shift_p = jnp.pad(shift.astype(jnp.float32), (0, _TBL - _NSPEC))
    mesh = plsc.VectorSubcoreMesh(core_axis_name="c", subcore_axis_name="s",
                                  num_cores=_NC, num_subcores=_NS)
    f = pl.kernel(
        functools.partial(_body, n, chunk),
        out_type=jax.ShapeDtypeStruct((n,), jnp.float32),
        mesh=mesh,
        interpret=interpret,
        compiler_params=pltpu.CompilerParams(needs_layout_passes=False),
        scratch_types=[
            pltpu.VMEM((_TBL,), jnp.float32),
            pltpu.VMEM((_TBL,), jnp.float32),
            pltpu.VMEM((chunk,), jnp.float32),
            pltpu.VMEM((chunk,), jnp.float32),
            pltpu.VMEM((chunk,), jnp.int32),
            pltpu.VMEM((chunk,), jnp.int32),
            pltpu.VMEM((chunk,), jnp.float32),
            pltpu.VMEM((chunk,), jnp.float32),
            pltpu.SemaphoreType.DMA,
            pltpu.SemaphoreType.DMA,
            pltpu.SemaphoreType.DMA,
            pltpu.SemaphoreType.DMA,
            pltpu.SemaphoreType.DMA,
            pltpu.SemaphoreType.DMA,
        ],
    )
    return f(x, Z.astype(jnp.int32), scale_p, shift_p)
